# baseline (speedup 1.0000x reference)
import functools

import jax
import jax.numpy as jnp
from jax import lax
from jax.experimental import pallas as pl
from jax.experimental.pallas import tpu as pltpu
from jax.experimental.pallas import tpu_sc as plsc

NC = 2
NS = 16
NW = NC * NS
L16 = 16


def _fsqrt(x):
    x = jnp.maximum(x, 1e-20)
    i = lax.bitcast_convert_type(x, jnp.int32)
    i = jnp.int32(0x5F3759DF) - lax.shift_right_logical(i, 1)
    y = lax.bitcast_convert_type(i, jnp.float32)
    for _ in range(3):
        y = y * (1.5 - 0.5 * x * y * y)
    return x * y


def _make_sc_kernel(B, S, D, DP, V):
    rows_w = (B // NW) * S
    n_chunks = rows_w // 128
    sent_w = B // NW
    n_sgrp = sent_w // L16
    idx_rows_w = rows_w // 128

    mesh = plsc.VectorSubcoreMesh(core_axis_name="c", subcore_axis_name="s")

    @functools.partial(
        pl.kernel,
        mesh=mesh,
        out_type=jax.ShapeDtypeStruct((B,), jnp.float32),
        compiler_params=pltpu.CompilerParams(
            use_tc_tiling_on_sc=False, needs_layout_passes=False),
        scratch_types=[
            pltpu.VMEM((idx_rows_w, 128), jnp.int32),
            pltpu.VMEM((rows_w, D), jnp.float32),
            pltpu.VMEM((DP,), jnp.float32),
            pltpu.VMEM((DP,), jnp.float32),
            pltpu.VMEM((rows_w,), jnp.float32),
            pltpu.VMEM((rows_w,), jnp.float32),
            pltpu.VMEM((sent_w,), jnp.float32),
            pltpu.SemaphoreType.DMA,
        ],
    )
    def sck(idx_hbm, table_hbm, u_hbm, w_hbm, out_hbm,
            idx_v, rows_v, u_v, w_v, alpha_v, awe_v, out_v, sem):
        wid = lax.axis_index("s") * NC + lax.axis_index("c")

        for j in range(n_chunks):
            pltpu.sync_copy(
                idx_hbm.at[pl.ds(wid * rows_w + j * 128, 128)], idx_v.at[j])
        pltpu.sync_copy(u_hbm, u_v)
        pltpu.sync_copy(w_hbm, w_v)

        handles = [
            pltpu.async_copy(table_hbm.at[idx_v.at[j]],
                             rows_v.at[pl.ds(j * 128, 128)], sem)
            for j in range(n_chunks)
        ]

        uacc = jnp.zeros((L16,), jnp.float32)
        for c in range(DP // L16):
            uc = u_v[pl.ds(c * L16, L16)]
            uacc = uacc + uc * uc
        u2 = jnp.sum(uacc)

        for h in handles:
            h.wait()

        lane = lax.iota(jnp.int32, L16)

        def group_body(g, _):
            row_idx = g * L16 + lane

            def d_body(dq, carry):
                accs = list(carry)
                for q in range(4):
                    d = dq * 4 + q
                    dcol = jnp.full((L16,), d, jnp.int32)
                    e = plsc.load_gather(rows_v, [row_idx, dcol])
                    u_b = plsc.load_gather(u_v, [dcol])
                    w_b = plsc.load_gather(w_v, [dcol])
                    accs[3 * q + 0] = accs[3 * q + 0] + e * u_b
                    accs[3 * q + 1] = accs[3 * q + 1] + e * w_b
                    accs[3 * q + 2] = accs[3 * q + 2] + e * e
                return tuple(accs)

            z = jnp.zeros((L16,), jnp.float32)
            accs = lax.fori_loop(0, D // 4, d_body, (z,) * 12)
            a_ue = accs[0] + accs[3] + accs[6] + accs[9]
            a_we = accs[1] + accs[4] + accs[7] + accs[10]
            a_e2 = accs[2] + accs[5] + accs[8] + accs[11]

            d2 = (u2 - 2.0 * a_ue) + a_e2
            alpha = jnp.exp(_fsqrt(d2))
            alpha_v[pl.ds(g * L16, L16)] = alpha
            awe_v[pl.ds(g * L16, L16)] = alpha * a_we
            return 0

        lax.fori_loop(0, rows_w // L16, group_body, 0)

        for sg in range(n_sgrp):
            srow = (sg * L16 + lane) * S

            def s_body(l, carry):
                acc_a, acc_aw = carry
                ridx = srow + l
                acc_a = acc_a + plsc.load_gather(alpha_v, [ridx])
                acc_aw = acc_aw + plsc.load_gather(awe_v, [ridx])
                return acc_a, acc_aw

            z = jnp.zeros((L16,), jnp.float32)
            acc_a, acc_aw = lax.fori_loop(0, S, s_body, (z, z))
            out_v[pl.ds(sg * L16, L16)] = acc_aw / jnp.maximum(acc_a, 1e-12)

        pltpu.sync_copy(out_v, out_hbm.at[pl.ds(wid * sent_w, sent_w)])

    return sck


def kernel(batch_word_idxs, word_embeddings, weights, attend_u):
    B, S = batch_word_idxs.shape
    V, D = word_embeddings.shape
    DP = ((D + L16 - 1) // L16) * L16
    idx_flat = batch_word_idxs.reshape(-1).astype(jnp.int32)
    u_pad = jnp.pad(attend_u.astype(jnp.float32), (0, DP - D))
    w_pad = jnp.pad(weights.reshape(-1).astype(jnp.float32), (0, DP - D))
    sck = _make_sc_kernel(B, S, D, DP, V)
    out = sck(idx_flat, word_embeddings.astype(jnp.float32), u_pad, w_pad)
    return out.reshape(B, 1)


# speedup vs baseline: 1.0269x; 1.0269x over previous
import functools

import jax
import jax.numpy as jnp
from jax import lax
from jax.experimental import pallas as pl
from jax.experimental.pallas import tpu as pltpu
from jax.experimental.pallas import tpu_sc as plsc

NC = 2
NS = 16
NW = NC * NS
L16 = 16
SR = 40
SUB = 3


def _fsqrt(x):
    x = jnp.maximum(x, 1e-20)
    i = lax.bitcast_convert_type(x, jnp.int32)
    i = jnp.int32(0x5F3759DF) - lax.shift_right_logical(i, 1)
    y = lax.bitcast_convert_type(i, jnp.float32)
    for _ in range(3):
        y = y * (1.5 - 0.5 * x * y * y)
    return x * y


def _make_sc_kernel(B, S, D, DP, V):
    assert D == 100, "sub-row cover scheme is specialized to D == 100"
    rows_w = (B // NW) * S
    n_chunks = rows_w // 128
    n_gchunks = rows_w * SUB // 128
    sent_w = B // NW
    n_sgrp = sent_w // L16

    mesh = plsc.VectorSubcoreMesh(core_axis_name="c", subcore_axis_name="s")

    @functools.partial(
        pl.kernel,
        mesh=mesh,
        out_type=jax.ShapeDtypeStruct((B,), jnp.float32),
        compiler_params=pltpu.CompilerParams(
            use_tc_tiling_on_sc=False, needs_layout_passes=False),
        scratch_types=[
            pltpu.VMEM((n_chunks, 128), jnp.int32),
            pltpu.VMEM((n_gchunks, 128), jnp.int32),
            pltpu.VMEM((rows_w * SUB, SR), jnp.float32),
            pltpu.VMEM((DP,), jnp.float32),
            pltpu.VMEM((DP,), jnp.float32),
            pltpu.VMEM((rows_w,), jnp.float32),
            pltpu.VMEM((rows_w,), jnp.float32),
            pltpu.VMEM((sent_w,), jnp.float32),
            pltpu.SemaphoreType.DMA,
        ],
    )
    def sck(idx_hbm, table_hbm, u_hbm, w_hbm, out_hbm,
            idx_v, gidx_v, rows_v, u_v, w_v, alpha_v, awe_v, out_v, sem):
        wid = lax.axis_index("s") * NC + lax.axis_index("c")

        for j in range(n_chunks):
            pltpu.sync_copy(
                idx_hbm.at[pl.ds(wid * rows_w + j * 128, 128)], idx_v.at[j])
        pltpu.sync_copy(u_hbm, u_v)
        pltpu.sync_copy(w_hbm, w_v)

        lane = lax.iota(jnp.int32, L16)

        for j in range(n_chunks):
            for c in range(128 // L16):
                v = idx_v[j, pl.ds(c * L16, L16)]
                r0 = 2 * v + lax.shift_right_logical(v, 1)
                p = 3 * (j * 128 + c * L16 + lane)
                for t in range(SUB):
                    pt = p + t
                    plsc.store_scatter(
                        gidx_v,
                        [lax.shift_right_logical(pt, 7), pt & 127],
                        r0 + t)

        handles = [
            pltpu.async_copy(table_hbm.at[gidx_v.at[j]],
                             rows_v.at[pl.ds(j * 128, 128)], sem)
            for j in range(n_gchunks)
        ]

        uacc = jnp.zeros((L16,), jnp.float32)
        for c in range(DP // L16):
            uc = u_v[pl.ds(c * L16, L16)]
            uacc = uacc + uc * uc
        u2 = jnp.sum(uacc)

        for h in handles:
            h.wait()

        lane3 = 3 * lane
        for g in range(rows_w // L16):
            widx = idx_v[g // 8, pl.ds((g % 8) * L16, L16)]
            off = (widx & 1) * 20
            row3 = lane3 + (3 * g * L16)

            def d_body(dq, carry, off=off, row3=row3):
                a_ue0, a_we0, a_e20, a_ue1, a_we1, a_e21 = carry
                accs = [a_ue0, a_we0, a_e20, a_ue1, a_we1, a_e21]
                for q in range(2):
                    d = dq * 2 + q
                    x = off + d
                    q40 = jnp.where(x >= 40, 1, 0) + jnp.where(x >= 80, 1, 0)
                    e = plsc.load_gather(rows_v, [row3 + q40, x - 40 * q40])
                    dcol = jnp.full((L16,), d, jnp.int32)
                    u_b = plsc.load_gather(u_v, [dcol])
                    w_b = plsc.load_gather(w_v, [dcol])
                    accs[3 * q + 0] = accs[3 * q + 0] + e * u_b
                    accs[3 * q + 1] = accs[3 * q + 1] + e * w_b
                    accs[3 * q + 2] = accs[3 * q + 2] + e * e
                return tuple(accs)

            z = jnp.zeros((L16,), jnp.float32)
            accs = lax.fori_loop(0, D // 2, d_body, (z,) * 6)
            a_ue = accs[0] + accs[3]
            a_we = accs[1] + accs[4]
            a_e2 = accs[2] + accs[5]

            d2 = (u2 - 2.0 * a_ue) + a_e2
            alpha = jnp.exp(_fsqrt(d2))
            alpha_v[pl.ds(g * L16, L16)] = alpha
            awe_v[pl.ds(g * L16, L16)] = alpha * a_we

        for sg in range(n_sgrp):
            srow = (sg * L16 + lane) * S

            def s_body(l, carry, srow=srow):
                acc_a, acc_aw = carry
                ridx = srow + l
                acc_a = acc_a + plsc.load_gather(alpha_v, [ridx])
                acc_aw = acc_aw + plsc.load_gather(awe_v, [ridx])
                return acc_a, acc_aw

            z = jnp.zeros((L16,), jnp.float32)
            acc_a, acc_aw = lax.fori_loop(0, S, s_body, (z, z))
            out_v[pl.ds(sg * L16, L16)] = acc_aw / jnp.maximum(acc_a, 1e-12)

        pltpu.sync_copy(out_v, out_hbm.at[pl.ds(wid * sent_w, sent_w)])

    return sck


def kernel(batch_word_idxs, word_embeddings, weights, attend_u):
    B, S = batch_word_idxs.shape
    V, D = word_embeddings.shape
    DP = ((D + L16 - 1) // L16) * L16
    idx_flat = batch_word_idxs.reshape(-1).astype(jnp.int32)
    table40 = word_embeddings.astype(jnp.float32).reshape(V * D // SR, SR)
    u_pad = jnp.pad(attend_u.astype(jnp.float32), (0, DP - D))
    w_pad = jnp.pad(weights.reshape(-1).astype(jnp.float32), (0, DP - D))
    sck = _make_sc_kernel(B, S, D, DP, V)
    out = sck(idx_flat, table40, u_pad, w_pad)
    return out.reshape(B, 1)


# speedup vs baseline: 1.4354x; 1.3977x over previous
import functools

import jax
import jax.numpy as jnp
from jax import lax
from jax.experimental import pallas as pl
from jax.experimental.pallas import tpu as pltpu
from jax.experimental.pallas import tpu_sc as plsc

NC = 2
NS = 16
NW = NC * NS
L16 = 16


def _fsqrt(x):
    x = jnp.maximum(x, 1e-20)
    i = lax.bitcast_convert_type(x, jnp.int32)
    i = jnp.int32(0x5F3759DF) - lax.shift_right_logical(i, 1)
    y = lax.bitcast_convert_type(i, jnp.float32)
    for _ in range(3):
        y = y * (1.5 - 0.5 * x * y * y)
    return x * y


def _make_tc_scan(V, D, BLK):
    def body(tab_ref, u_ref, w_ref, su_ref, sw_ref, se2_ref):
        e = tab_ref[...]
        ub = u_ref[...]
        wb = w_ref[...]
        su_ref[...] = jnp.sum(e * ub[None, :], axis=1)
        sw_ref[...] = jnp.sum(e * wb[None, :], axis=1)
        se2_ref[...] = jnp.sum(e * e, axis=1)

    grid = (V + BLK - 1) // BLK
    return pl.pallas_call(
        body,
        grid=(grid,),
        in_specs=[
            pl.BlockSpec((BLK, D), lambda i: (i, 0)),
            pl.BlockSpec((D,), lambda i: (0,)),
            pl.BlockSpec((D,), lambda i: (0,)),
        ],
        out_specs=[
            pl.BlockSpec((BLK,), lambda i: (i,)),
            pl.BlockSpec((BLK,), lambda i: (i,)),
            pl.BlockSpec((BLK,), lambda i: (i,)),
        ],
        out_shape=[jax.ShapeDtypeStruct((V,), jnp.float32)] * 3,
    )


def _make_sc_kernel(B, S, D, DP, V):
    rows_w = (B // NW) * S
    n_chunks = rows_w // 128
    sent_w = B // NW
    n_sgrp = sent_w // L16

    mesh = plsc.VectorSubcoreMesh(core_axis_name="c", subcore_axis_name="s")

    @functools.partial(
        pl.kernel,
        mesh=mesh,
        out_type=jax.ShapeDtypeStruct((B,), jnp.float32),
        compiler_params=pltpu.CompilerParams(
            use_tc_tiling_on_sc=False, needs_layout_passes=False),
        scratch_types=[
            pltpu.VMEM((n_chunks, 128), jnp.int32),
            pltpu.VMEM((rows_w,), jnp.float32),
            pltpu.VMEM((rows_w,), jnp.float32),
            pltpu.VMEM((rows_w,), jnp.float32),
            pltpu.VMEM((DP,), jnp.float32),
            pltpu.VMEM((rows_w,), jnp.float32),
            pltpu.VMEM((rows_w,), jnp.float32),
            pltpu.VMEM((sent_w,), jnp.float32),
            pltpu.SemaphoreType.DMA,
        ],
    )
    def sck(idx_hbm, su_hbm, sw_hbm, se2_hbm, u_hbm, out_hbm,
            idx_v, su_v, sw_v, se2_v, u_v, alpha_v, awe_v, out_v, sem):
        wid = lax.axis_index("s") * NC + lax.axis_index("c")

        for j in range(n_chunks):
            pltpu.sync_copy(
                idx_hbm.at[pl.ds(wid * rows_w + j * 128, 128)], idx_v.at[j])
        pltpu.sync_copy(u_hbm, u_v)

        handles = []
        for j in range(n_chunks):
            sl = pl.ds(j * 128, 128)
            handles.append(
                pltpu.async_copy(su_hbm.at[idx_v.at[j]], su_v.at[sl], sem))
            handles.append(
                pltpu.async_copy(sw_hbm.at[idx_v.at[j]], sw_v.at[sl], sem))
            handles.append(
                pltpu.async_copy(se2_hbm.at[idx_v.at[j]], se2_v.at[sl], sem))

        uacc = jnp.zeros((L16,), jnp.float32)
        for c in range(DP // L16):
            uc = u_v[pl.ds(c * L16, L16)]
            uacc = uacc + uc * uc
        u2 = jnp.sum(uacc)

        for h in handles:
            h.wait()

        for g in range(rows_w // L16):
            sl = pl.ds(g * L16, L16)
            su = su_v[sl]
            sw = sw_v[sl]
            se2 = se2_v[sl]
            d2 = (u2 - 2.0 * su) + se2
            alpha = jnp.exp(_fsqrt(d2))
            alpha_v[sl] = alpha
            awe_v[sl] = alpha * sw

        lane = lax.iota(jnp.int32, L16)
        for sg in range(n_sgrp):
            srow = (sg * L16 + lane) * S

            def s_body(l, carry, srow=srow):
                acc_a, acc_aw = carry
                ridx = srow + l
                acc_a = acc_a + plsc.load_gather(alpha_v, [ridx])
                acc_aw = acc_aw + plsc.load_gather(awe_v, [ridx])
                return acc_a, acc_aw

            z = jnp.zeros((L16,), jnp.float32)
            acc_a, acc_aw = lax.fori_loop(0, S, s_body, (z, z))
            out_v[pl.ds(sg * L16, L16)] = acc_aw / jnp.maximum(acc_a, 1e-12)

        pltpu.sync_copy(out_v, out_hbm.at[pl.ds(wid * sent_w, sent_w)])

    return sck


def kernel(batch_word_idxs, word_embeddings, weights, attend_u):
    B, S = batch_word_idxs.shape
    V, D = word_embeddings.shape
    DP = ((D + L16 - 1) // L16) * L16
    idx_flat = batch_word_idxs.reshape(-1).astype(jnp.int32)
    table = word_embeddings.astype(jnp.float32)
    u_vec = attend_u.astype(jnp.float32)
    w_vec = weights.reshape(-1).astype(jnp.float32)
    u_pad = jnp.pad(u_vec, (0, DP - D))

    su, sw, se2 = _make_tc_scan(V, D, 1024)(table, u_vec, w_vec)
    sck = _make_sc_kernel(B, S, D, DP, V)
    out = sck(idx_flat, su, sw, se2, u_pad)
    return out.reshape(B, 1)


# speedup vs baseline: 1.9075x; 1.3289x over previous
import functools

import jax
import jax.numpy as jnp
from jax import lax
from jax.experimental import pallas as pl
from jax.experimental.pallas import tpu as pltpu
from jax.experimental.pallas import tpu_sc as plsc

NC = 2
NS = 16
NW = NC * NS
L16 = 16
PK = 128


def _fsqrt(x):
    x = jnp.maximum(x, 1e-20)
    i = lax.bitcast_convert_type(x, jnp.int32)
    i = jnp.int32(0x5F3759DF) - lax.shift_right_logical(i, 1)
    y = lax.bitcast_convert_type(i, jnp.float32)
    for _ in range(3):
        y = y * (1.5 - 0.5 * x * y * y)
    return x * y


def _make_tc_scan(V, D, BLK):
    def body(tab_ref, m1_ref, m2_ref, out_ref):
        e = tab_ref[...]
        f = e * e
        out_ref[...] = (
            jnp.dot(e, m1_ref[...], preferred_element_type=jnp.float32)
            + jnp.dot(f, m2_ref[...], preferred_element_type=jnp.float32))

    grid = (V + BLK - 1) // BLK
    return pl.pallas_call(
        body,
        grid=(grid,),
        in_specs=[
            pl.BlockSpec((BLK, D), lambda i: (i, 0)),
            pl.BlockSpec((D, PK), lambda i: (0, 0)),
            pl.BlockSpec((D, PK), lambda i: (0, 0)),
        ],
        out_specs=pl.BlockSpec((BLK, PK), lambda i: (i, 0)),
        out_shape=jax.ShapeDtypeStruct((grid * BLK, PK), jnp.float32),
    )


def _make_sc_kernel(B, S, D, DP, VP):
    rows_w = (B // NW) * S
    n_chunks = rows_w // 128
    sent_w = B // NW
    n_sgrp = sent_w // L16

    mesh = plsc.VectorSubcoreMesh(core_axis_name="c", subcore_axis_name="s")

    @functools.partial(
        pl.kernel,
        mesh=mesh,
        out_type=jax.ShapeDtypeStruct((B,), jnp.float32),
        compiler_params=pltpu.CompilerParams(
            use_tc_tiling_on_sc=False, needs_layout_passes=False),
        scratch_types=[
            pltpu.VMEM((n_chunks, 128), jnp.int32),
            pltpu.VMEM((rows_w, PK), jnp.float32),
            pltpu.VMEM((DP,), jnp.float32),
            pltpu.VMEM((rows_w,), jnp.float32),
            pltpu.VMEM((rows_w,), jnp.float32),
            pltpu.VMEM((sent_w,), jnp.float32),
            pltpu.SemaphoreType.DMA,
        ],
    )
    def sck(idx_hbm, pk_hbm, u_hbm, out_hbm,
            idx_v, pk_v, u_v, alpha_v, awe_v, out_v, sem):
        wid = lax.axis_index("s") * NC + lax.axis_index("c")

        for j in range(n_chunks):
            pltpu.sync_copy(
                idx_hbm.at[pl.ds(wid * rows_w + j * 128, 128)], idx_v.at[j])
        pltpu.sync_copy(u_hbm, u_v)

        handles = [
            pltpu.async_copy(pk_hbm.at[idx_v.at[j]],
                             pk_v.at[pl.ds(j * 128, 128)], sem)
            for j in range(n_chunks)
        ]

        uacc = jnp.zeros((L16,), jnp.float32)
        for c in range(DP // L16):
            uc = u_v[pl.ds(c * L16, L16)]
            uacc = uacc + uc * uc
        u2 = jnp.sum(uacc)

        for h in handles:
            h.wait()

        lane = lax.iota(jnp.int32, L16)
        zc = jnp.zeros((L16,), jnp.int32)
        for g in range(rows_w // L16):
            row_idx = g * L16 + lane
            su = plsc.load_gather(pk_v, [row_idx, zc])
            sw = plsc.load_gather(pk_v, [row_idx, zc + 1])
            se2 = plsc.load_gather(pk_v, [row_idx, zc + 2])
            d2 = (u2 - 2.0 * su) + se2
            alpha = jnp.exp(_fsqrt(d2))
            sl = pl.ds(g * L16, L16)
            alpha_v[sl] = alpha
            awe_v[sl] = alpha * sw

        for sg in range(n_sgrp):
            srow = (sg * L16 + lane) * S

            def s_body(l, carry, srow=srow):
                acc_a, acc_aw = carry
                ridx = srow + l
                acc_a = acc_a + plsc.load_gather(alpha_v, [ridx])
                acc_aw = acc_aw + plsc.load_gather(awe_v, [ridx])
                return acc_a, acc_aw

            z = jnp.zeros((L16,), jnp.float32)
            acc_a, acc_aw = lax.fori_loop(0, S, s_body, (z, z))
            out_v[pl.ds(sg * L16, L16)] = acc_aw / jnp.maximum(acc_a, 1e-12)

        pltpu.sync_copy(out_v, out_hbm.at[pl.ds(wid * sent_w, sent_w)])

    return sck


def kernel(batch_word_idxs, word_embeddings, weights, attend_u):
    B, S = batch_word_idxs.shape
    V, D = word_embeddings.shape
    DP = ((D + L16 - 1) // L16) * L16
    idx_flat = batch_word_idxs.reshape(-1).astype(jnp.int32)
    table = word_embeddings.astype(jnp.float32)
    u_vec = attend_u.astype(jnp.float32)
    w_vec = weights.reshape(-1).astype(jnp.float32)
    u_pad = jnp.pad(u_vec, (0, DP - D))

    m1 = jnp.zeros((D, PK), jnp.float32)
    m1 = m1.at[:, 0].set(u_vec).at[:, 1].set(w_vec)
    m2 = jnp.zeros((D, PK), jnp.float32).at[:, 2].set(1.0)

    pk = _make_tc_scan(V, D, 1024)(table, m1, m2)
    sck = _make_sc_kernel(B, S, D, DP, pk.shape[0])
    out = sck(idx_flat, pk, u_pad)
    return out.reshape(B, 1)


# speedup vs baseline: 1.9807x; 1.0384x over previous
import functools

import jax
import jax.numpy as jnp
from jax import lax
from jax.experimental import pallas as pl
from jax.experimental.pallas import tpu as pltpu
from jax.experimental.pallas import tpu_sc as plsc

NC = 2
NS = 16
NW = NC * NS
L16 = 16
PK = 128


def _fsqrt(x):
    x = jnp.maximum(x, 1e-20)
    i = lax.bitcast_convert_type(x, jnp.int32)
    i = jnp.int32(0x5F3759DF) - lax.shift_right_logical(i, 1)
    y = lax.bitcast_convert_type(i, jnp.float32)
    for _ in range(3):
        y = y * (1.5 - 0.5 * x * y * y)
    return x * y


def _make_tc_scan(V, D, BLK):
    def body(tab_ref, m1_ref, m2_ref, out_ref):
        e = tab_ref[...]
        f = e * e
        s = (jnp.dot(e, m1_ref[...], preferred_element_type=jnp.float32)
             + jnp.dot(f, m2_ref[...], preferred_element_type=jnp.float32))
        out_ref[...] = s[:, 0:8].T

    grid = (V + BLK - 1) // BLK
    return pl.pallas_call(
        body,
        grid=(grid,),
        in_specs=[
            pl.BlockSpec((BLK, D), lambda i: (i, 0)),
            pl.BlockSpec((D, PK), lambda i: (0, 0)),
            pl.BlockSpec((D, PK), lambda i: (0, 0)),
        ],
        out_specs=pl.BlockSpec((8, BLK), lambda i: (i, 0)),
        out_shape=jax.ShapeDtypeStruct((grid * 8, BLK), jnp.float32),
    )


def _make_sc_kernel(B, S, D, DP, VP):
    rows_w = (B // NW) * S
    n_chunks = rows_w // 128
    sent_w = B // NW
    n_sgrp = sent_w // L16

    mesh = plsc.VectorSubcoreMesh(core_axis_name="c", subcore_axis_name="s")

    @functools.partial(
        pl.kernel,
        mesh=mesh,
        out_type=jax.ShapeDtypeStruct((B,), jnp.float32),
        compiler_params=pltpu.CompilerParams(
            use_tc_tiling_on_sc=False, needs_layout_passes=False),
        scratch_types=[
            pltpu.VMEM((n_chunks, 128), jnp.int32),
            pltpu.VMEM((3 * n_chunks, 128), jnp.int32),
            pltpu.VMEM((3 * rows_w,), jnp.float32),
            pltpu.VMEM((DP,), jnp.float32),
            pltpu.VMEM((rows_w,), jnp.float32),
            pltpu.VMEM((rows_w,), jnp.float32),
            pltpu.VMEM((sent_w,), jnp.float32),
            pltpu.SemaphoreType.DMA,
        ],
    )
    def sck(idx_hbm, pk_hbm, u_hbm, out_hbm,
            idx_v, gidx_v, pk_v, u_v, alpha_v, awe_v, out_v, sem):
        wid = lax.axis_index("s") * NC + lax.axis_index("c")

        for j in range(n_chunks):
            pltpu.sync_copy(
                idx_hbm.at[pl.ds(wid * rows_w + j * 128, 128)], idx_v.at[j])
        pltpu.sync_copy(u_hbm, u_v)

        lane = lax.iota(jnp.int32, L16)

        for j in range(n_chunks):
            for c in range(128 // L16):
                v = idx_v[j, pl.ds(c * L16, L16)]
                base = (lax.shift_left(lax.shift_right_logical(v, 10), 13)
                        + (v & 1023))
                p = 3 * (j * 128 + c * L16 + lane)
                for t in range(3):
                    pt = p + t
                    plsc.store_scatter(
                        gidx_v,
                        [lax.shift_right_logical(pt, 7), pt & 127],
                        base + 1024 * t)

        handles = [
            pltpu.async_copy(pk_hbm.at[gidx_v.at[j]],
                             pk_v.at[pl.ds(j * 128, 128)], sem)
            for j in range(3 * n_chunks)
        ]

        uacc = jnp.zeros((L16,), jnp.float32)
        for c in range(DP // L16):
            uc = u_v[pl.ds(c * L16, L16)]
            uacc = uacc + uc * uc
        u2 = jnp.sum(uacc)

        for h in handles:
            h.wait()

        lane3 = 3 * lane
        for g in range(rows_w // L16):
            base3 = lane3 + 3 * g * L16
            su = plsc.load_gather(pk_v, [base3])
            sw = plsc.load_gather(pk_v, [base3 + 1])
            se2 = plsc.load_gather(pk_v, [base3 + 2])
            d2 = (u2 - 2.0 * su) + se2
            alpha = jnp.exp(_fsqrt(d2))
            sl = pl.ds(g * L16, L16)
            alpha_v[sl] = alpha
            awe_v[sl] = alpha * sw

        for sg in range(n_sgrp):
            srow = (sg * L16 + lane) * S

            def s_body(l, carry, srow=srow):
                acc_a, acc_aw = carry
                ridx = srow + l
                acc_a = acc_a + plsc.load_gather(alpha_v, [ridx])
                acc_aw = acc_aw + plsc.load_gather(awe_v, [ridx])
                return acc_a, acc_aw

            z = jnp.zeros((L16,), jnp.float32)
            acc_a, acc_aw = lax.fori_loop(0, S, s_body, (z, z))
            out_v[pl.ds(sg * L16, L16)] = acc_aw / jnp.maximum(acc_a, 1e-12)

        pltpu.sync_copy(out_v, out_hbm.at[pl.ds(wid * sent_w, sent_w)])

    return sck


def kernel(batch_word_idxs, word_embeddings, weights, attend_u):
    B, S = batch_word_idxs.shape
    V, D = word_embeddings.shape
    DP = ((D + L16 - 1) // L16) * L16
    idx_flat = batch_word_idxs.reshape(-1).astype(jnp.int32)
    table = word_embeddings.astype(jnp.float32)
    u_vec = attend_u.astype(jnp.float32)
    w_vec = weights.reshape(-1).astype(jnp.float32)
    u_pad = jnp.pad(u_vec, (0, DP - D))

    m1 = jnp.zeros((D, PK), jnp.float32)
    m1 = m1.at[:, 0].set(u_vec).at[:, 1].set(w_vec)
    m2 = jnp.zeros((D, PK), jnp.float32).at[:, 2].set(1.0)

    pk = _make_tc_scan(V, D, 1024)(table, m1, m2).reshape(-1)
    sck = _make_sc_kernel(B, S, D, DP, pk.shape[0])
    out = sck(idx_flat, pk, u_pad)
    return out.reshape(B, 1)


# speedup vs baseline: 2.3925x; 1.2079x over previous
import functools

import jax
import jax.numpy as jnp
from jax import lax
from jax.experimental import pallas as pl
from jax.experimental.pallas import tpu as pltpu
from jax.experimental.pallas import tpu_sc as plsc

NC = 2
NS = 16
NW = NC * NS
L16 = 16
PK = 128
BLK = 2048
BLK_LOG = 11


def _fsqrt(x):
    x = jnp.maximum(x, 1e-20)
    i = lax.bitcast_convert_type(x, jnp.int32)
    i = jnp.int32(0x5F3759DF) - lax.shift_right_logical(i, 1)
    y = lax.bitcast_convert_type(i, jnp.float32)
    for _ in range(3):
        y = y * (1.5 - 0.5 * x * y * y)
    return x * y


def _make_tc_scan(V, D):
    def body(tab_ref, m1_ref, m2_ref, out_ref):
        e = tab_ref[...]
        f = e * e
        s = (jnp.dot(e, m1_ref[...], preferred_element_type=jnp.float32)
             + jnp.dot(f, m2_ref[...], preferred_element_type=jnp.float32))
        out_ref[...] = s[:, 0:8].T

    grid = (V + BLK - 1) // BLK
    return pl.pallas_call(
        body,
        grid=(grid,),
        in_specs=[
            pl.BlockSpec((BLK, D), lambda i: (i, 0)),
            pl.BlockSpec((D, PK), lambda i: (0, 0)),
            pl.BlockSpec((D, PK), lambda i: (0, 0)),
        ],
        out_specs=pl.BlockSpec((8, BLK), lambda i: (i, 0)),
        out_shape=jax.ShapeDtypeStruct((grid * 8, BLK), jnp.float32),
    )


def _make_sc_kernel(B, S, D, DP, VP):
    rows_w = (B // NW) * S
    n_chunks = rows_w // 128
    sent_w = B // NW
    n_sgrp = sent_w // L16

    mesh = plsc.VectorSubcoreMesh(core_axis_name="c", subcore_axis_name="s")

    @functools.partial(
        pl.kernel,
        mesh=mesh,
        out_type=jax.ShapeDtypeStruct((B,), jnp.float32),
        compiler_params=pltpu.CompilerParams(
            use_tc_tiling_on_sc=False, needs_layout_passes=False),
        scratch_types=[
            pltpu.VMEM((n_chunks, 128), jnp.int32),
            pltpu.VMEM((3 * n_chunks, 128), jnp.int32),
            pltpu.VMEM((3 * rows_w,), jnp.float32),
            pltpu.VMEM((DP,), jnp.float32),
            pltpu.VMEM((rows_w,), jnp.float32),
            pltpu.VMEM((rows_w,), jnp.float32),
            pltpu.VMEM((sent_w,), jnp.float32),
            pltpu.SemaphoreType.DMA,
        ],
    )
    def sck(idx_hbm, pk_hbm, u_hbm, out_hbm,
            idx_v, gidx_v, pk_v, u_v, alpha_v, awe_v, out_v, sem):
        wid = lax.axis_index("s") * NC + lax.axis_index("c")

        for j in range(n_chunks):
            pltpu.sync_copy(
                idx_hbm.at[pl.ds(wid * rows_w + j * 128, 128)], idx_v.at[j])
        pltpu.sync_copy(u_hbm, u_v)

        lane = lax.iota(jnp.int32, L16)

        for j in range(n_chunks):
            for c in range(128 // L16):
                v = idx_v[j, pl.ds(c * L16, L16)]
                base = (lax.shift_left(
                    lax.shift_right_logical(v, BLK_LOG), BLK_LOG + 3)
                    + (v & (BLK - 1)))
                p = 3 * (j * 128 + c * L16 + lane)
                for t in range(3):
                    pt = p + t
                    plsc.store_scatter(
                        gidx_v,
                        [lax.shift_right_logical(pt, 7), pt & 127],
                        base + BLK * t)

        handles = [
            pltpu.async_copy(pk_hbm.at[gidx_v.at[j]],
                             pk_v.at[pl.ds(j * 128, 128)], sem)
            for j in range(3 * n_chunks)
        ]

        uacc = jnp.zeros((L16,), jnp.float32)
        for c in range(DP // L16):
            uc = u_v[pl.ds(c * L16, L16)]
            uacc = uacc + uc * uc
        u2 = jnp.sum(uacc)

        for h in handles:
            h.wait()

        lane3 = 3 * lane
        for g in range(rows_w // L16):
            base3 = lane3 + 3 * g * L16
            su = plsc.load_gather(pk_v, [base3])
            sw = plsc.load_gather(pk_v, [base3 + 1])
            se2 = plsc.load_gather(pk_v, [base3 + 2])
            d2 = (u2 - 2.0 * su) + se2
            alpha = jnp.exp(_fsqrt(d2))
            sl = pl.ds(g * L16, L16)
            alpha_v[sl] = alpha
            awe_v[sl] = alpha * sw

        for sg in range(n_sgrp):
            srow = (sg * L16 + lane) * S

            def s_body(l, carry, srow=srow):
                acc_a, acc_aw = carry
                ridx = srow + l
                acc_a = acc_a + plsc.load_gather(alpha_v, [ridx])
                acc_aw = acc_aw + plsc.load_gather(awe_v, [ridx])
                return acc_a, acc_aw

            z = jnp.zeros((L16,), jnp.float32)
            acc_a, acc_aw = lax.fori_loop(0, S, s_body, (z, z))
            out_v[pl.ds(sg * L16, L16)] = acc_aw / jnp.maximum(acc_a, 1e-12)

        pltpu.sync_copy(out_v, out_hbm.at[pl.ds(wid * sent_w, sent_w)])

    return sck


def kernel(batch_word_idxs, word_embeddings, weights, attend_u):
    B, S = batch_word_idxs.shape
    V, D = word_embeddings.shape
    DP = ((D + L16 - 1) // L16) * L16
    idx_flat = batch_word_idxs.reshape(-1).astype(jnp.int32)
    table = word_embeddings.astype(jnp.float32)
    u_vec = attend_u.astype(jnp.float32)
    w_vec = weights.reshape(-1).astype(jnp.float32)
    u_pad = jnp.pad(u_vec, (0, DP - D))

    m1 = jnp.zeros((D, PK), jnp.float32)
    m1 = m1.at[:, 0].set(u_vec).at[:, 1].set(w_vec)
    m2 = jnp.zeros((D, PK), jnp.float32).at[:, 2].set(1.0)

    pk = _make_tc_scan(V, D)(table, m1, m2).reshape(-1)
    sck = _make_sc_kernel(B, S, D, DP, pk.shape[0])
    out = sck(idx_flat, pk, u_pad)
    return out.reshape(B, 1)


# speedup vs baseline: 2.6744x; 1.1179x over previous
import functools

import jax
import jax.numpy as jnp
from jax import lax
from jax.experimental import pallas as pl
from jax.experimental.pallas import tpu as pltpu
from jax.experimental.pallas import tpu_sc as plsc

NC = 2
NS = 16
NW = NC * NS
L16 = 16
PK = 128
BLK = 4096
BLK_LOG = 12


def _fsqrt(x):
    x = jnp.maximum(x, 1e-20)
    i = lax.bitcast_convert_type(x, jnp.int32)
    i = jnp.int32(0x5F3759DF) - lax.shift_right_logical(i, 1)
    y = lax.bitcast_convert_type(i, jnp.float32)
    for _ in range(3):
        y = y * (1.5 - 0.5 * x * y * y)
    return x * y


def _make_tc_scan(V, D):
    def body(tab_ref, m1_ref, m2_ref, out_ref):
        e = tab_ref[...]
        f = e * e
        s = (jnp.dot(e, m1_ref[...], preferred_element_type=jnp.float32)
             + jnp.dot(f, m2_ref[...], preferred_element_type=jnp.float32))
        out_ref[...] = s[:, 0:8].T

    grid = (V + BLK - 1) // BLK
    return pl.pallas_call(
        body,
        grid=(grid,),
        in_specs=[
            pl.BlockSpec((BLK, D), lambda i: (i, 0)),
            pl.BlockSpec((D, PK), lambda i: (0, 0)),
            pl.BlockSpec((D, PK), lambda i: (0, 0)),
        ],
        out_specs=pl.BlockSpec((8, BLK), lambda i: (i, 0)),
        out_shape=jax.ShapeDtypeStruct((grid * 8, BLK), jnp.float32),
    )


def _make_sc_kernel(B, S, D, DP, VP):
    rows_w = (B // NW) * S
    n_chunks = rows_w // 128
    sent_w = B // NW
    n_sgrp = sent_w // L16

    mesh = plsc.VectorSubcoreMesh(core_axis_name="c", subcore_axis_name="s")

    @functools.partial(
        pl.kernel,
        mesh=mesh,
        out_type=jax.ShapeDtypeStruct((B,), jnp.float32),
        compiler_params=pltpu.CompilerParams(
            use_tc_tiling_on_sc=False, needs_layout_passes=False),
        scratch_types=[
            pltpu.VMEM((n_chunks, 128), jnp.int32),
            pltpu.VMEM((3 * n_chunks, 128), jnp.int32),
            pltpu.VMEM((3 * rows_w,), jnp.float32),
            pltpu.VMEM((DP,), jnp.float32),
            pltpu.VMEM((rows_w,), jnp.float32),
            pltpu.VMEM((rows_w,), jnp.float32),
            pltpu.VMEM((sent_w,), jnp.float32),
            pltpu.SemaphoreType.DMA,
        ],
    )
    def sck(idx_hbm, pk_hbm, u_hbm, out_hbm,
            idx_v, gidx_v, pk_v, u_v, alpha_v, awe_v, out_v, sem):
        wid = lax.axis_index("s") * NC + lax.axis_index("c")

        for j in range(n_chunks):
            pltpu.sync_copy(
                idx_hbm.at[pl.ds(wid * rows_w + j * 128, 128)], idx_v.at[j])
        pltpu.sync_copy(u_hbm, u_v)

        lane = lax.iota(jnp.int32, L16)

        for j in range(n_chunks):
            for c in range(128 // L16):
                v = idx_v[j, pl.ds(c * L16, L16)]
                base = (lax.shift_left(
                    lax.shift_right_logical(v, BLK_LOG), BLK_LOG + 3)
                    + (v & (BLK - 1)))
                p = 3 * (j * 128 + c * L16 + lane)
                for t in range(3):
                    pt = p + t
                    plsc.store_scatter(
                        gidx_v,
                        [lax.shift_right_logical(pt, 7), pt & 127],
                        base + BLK * t)

        handles = [
            pltpu.async_copy(pk_hbm.at[gidx_v.at[j]],
                             pk_v.at[pl.ds(j * 128, 128)], sem)
            for j in range(3 * n_chunks)
        ]

        uacc = jnp.zeros((L16,), jnp.float32)
        for c in range(DP // L16):
            uc = u_v[pl.ds(c * L16, L16)]
            uacc = uacc + uc * uc
        u2 = jnp.sum(uacc)

        for h in handles:
            h.wait()

        lane3 = 3 * lane
        for g in range(rows_w // L16):
            base3 = lane3 + 3 * g * L16
            su = plsc.load_gather(pk_v, [base3])
            sw = plsc.load_gather(pk_v, [base3 + 1])
            se2 = plsc.load_gather(pk_v, [base3 + 2])
            d2 = (u2 - 2.0 * su) + se2
            alpha = jnp.exp(_fsqrt(d2))
            sl = pl.ds(g * L16, L16)
            alpha_v[sl] = alpha
            awe_v[sl] = alpha * sw

        for sg in range(n_sgrp):
            srow = (sg * L16 + lane) * S

            def s_body(l, carry, srow=srow):
                acc_a, acc_aw = carry
                ridx = srow + l
                acc_a = acc_a + plsc.load_gather(alpha_v, [ridx])
                acc_aw = acc_aw + plsc.load_gather(awe_v, [ridx])
                return acc_a, acc_aw

            z = jnp.zeros((L16,), jnp.float32)
            acc_a, acc_aw = lax.fori_loop(0, S, s_body, (z, z))
            out_v[pl.ds(sg * L16, L16)] = acc_aw / jnp.maximum(acc_a, 1e-12)

        pltpu.sync_copy(out_v, out_hbm.at[pl.ds(wid * sent_w, sent_w)])

    return sck


def kernel(batch_word_idxs, word_embeddings, weights, attend_u):
    B, S = batch_word_idxs.shape
    V, D = word_embeddings.shape
    DP = ((D + L16 - 1) // L16) * L16
    idx_flat = batch_word_idxs.reshape(-1).astype(jnp.int32)
    table = word_embeddings.astype(jnp.float32)
    u_vec = attend_u.astype(jnp.float32)
    w_vec = weights.reshape(-1).astype(jnp.float32)
    u_pad = jnp.pad(u_vec, (0, DP - D))

    m1 = jnp.zeros((D, PK), jnp.float32)
    m1 = m1.at[:, 0].set(u_vec).at[:, 1].set(w_vec)
    m2 = jnp.zeros((D, PK), jnp.float32).at[:, 2].set(1.0)

    pk = _make_tc_scan(V, D)(table, m1, m2).reshape(-1)
    sck = _make_sc_kernel(B, S, D, DP, pk.shape[0])
    out = sck(idx_flat, pk, u_pad)
    return out.reshape(B, 1)


# speedup vs baseline: 2.8443x; 1.0635x over previous
import functools

import jax
import jax.numpy as jnp
from jax import lax
from jax.experimental import pallas as pl
from jax.experimental.pallas import tpu as pltpu
from jax.experimental.pallas import tpu_sc as plsc

NC = 2
NS = 16
NW = NC * NS
L16 = 16
PK = 128
BLK = 8192
BLK_LOG = 13


def _fsqrt(x):
    x = jnp.maximum(x, 1e-20)
    i = lax.bitcast_convert_type(x, jnp.int32)
    i = jnp.int32(0x5F3759DF) - lax.shift_right_logical(i, 1)
    y = lax.bitcast_convert_type(i, jnp.float32)
    for _ in range(3):
        y = y * (1.5 - 0.5 * x * y * y)
    return x * y


def _make_tc_scan(V, D):
    def body(tab_ref, m1_ref, m2_ref, out_ref):
        e = tab_ref[...]
        f = e * e
        s = (jnp.dot(e, m1_ref[...], preferred_element_type=jnp.float32)
             + jnp.dot(f, m2_ref[...], preferred_element_type=jnp.float32))
        out_ref[...] = s[:, 0:8].T

    grid = (V + BLK - 1) // BLK
    return pl.pallas_call(
        body,
        grid=(grid,),
        in_specs=[
            pl.BlockSpec((BLK, D), lambda i: (i, 0)),
            pl.BlockSpec((D, PK), lambda i: (0, 0)),
            pl.BlockSpec((D, PK), lambda i: (0, 0)),
        ],
        out_specs=pl.BlockSpec((8, BLK), lambda i: (i, 0)),
        out_shape=jax.ShapeDtypeStruct((grid * 8, BLK), jnp.float32),
    )


def _make_sc_kernel(B, S, D, DP, VP):
    rows_w = (B // NW) * S
    n_chunks = rows_w // 128
    sent_w = B // NW
    n_sgrp = sent_w // L16

    mesh = plsc.VectorSubcoreMesh(core_axis_name="c", subcore_axis_name="s")

    @functools.partial(
        pl.kernel,
        mesh=mesh,
        out_type=jax.ShapeDtypeStruct((B,), jnp.float32),
        compiler_params=pltpu.CompilerParams(
            use_tc_tiling_on_sc=False, needs_layout_passes=False),
        scratch_types=[
            pltpu.VMEM((n_chunks, 128), jnp.int32),
            pltpu.VMEM((3 * n_chunks, 128), jnp.int32),
            pltpu.VMEM((3 * rows_w,), jnp.float32),
            pltpu.VMEM((DP,), jnp.float32),
            pltpu.VMEM((rows_w,), jnp.float32),
            pltpu.VMEM((rows_w,), jnp.float32),
            pltpu.VMEM((sent_w,), jnp.float32),
            pltpu.SemaphoreType.DMA,
        ],
    )
    def sck(idx_hbm, pk_hbm, u_hbm, out_hbm,
            idx_v, gidx_v, pk_v, u_v, alpha_v, awe_v, out_v, sem):
        wid = lax.axis_index("s") * NC + lax.axis_index("c")

        for j in range(n_chunks):
            pltpu.sync_copy(
                idx_hbm.at[pl.ds(wid * rows_w + j * 128, 128)], idx_v.at[j])
        pltpu.sync_copy(u_hbm, u_v)

        lane = lax.iota(jnp.int32, L16)

        for j in range(n_chunks):
            for c in range(128 // L16):
                v = idx_v[j, pl.ds(c * L16, L16)]
                base = (lax.shift_left(
                    lax.shift_right_logical(v, BLK_LOG), BLK_LOG + 3)
                    + (v & (BLK - 1)))
                p = 3 * (j * 128 + c * L16 + lane)
                for t in range(3):
                    pt = p + t
                    plsc.store_scatter(
                        gidx_v,
                        [lax.shift_right_logical(pt, 7), pt & 127],
                        base + BLK * t)

        handles = [
            pltpu.async_copy(pk_hbm.at[gidx_v.at[j]],
                             pk_v.at[pl.ds(j * 128, 128)], sem)
            for j in range(3 * n_chunks)
        ]

        uacc = jnp.zeros((L16,), jnp.float32)
        for c in range(DP // L16):
            uc = u_v[pl.ds(c * L16, L16)]
            uacc = uacc + uc * uc
        u2 = jnp.sum(uacc)

        for h in handles:
            h.wait()

        lane3 = 3 * lane
        for g in range(rows_w // L16):
            base3 = lane3 + 3 * g * L16
            su = plsc.load_gather(pk_v, [base3])
            sw = plsc.load_gather(pk_v, [base3 + 1])
            se2 = plsc.load_gather(pk_v, [base3 + 2])
            d2 = (u2 - 2.0 * su) + se2
            alpha = jnp.exp(_fsqrt(d2))
            sl = pl.ds(g * L16, L16)
            alpha_v[sl] = alpha
            awe_v[sl] = alpha * sw

        for sg in range(n_sgrp):
            srow = (sg * L16 + lane) * S

            def s_body(l, carry, srow=srow):
                acc_a, acc_aw = carry
                ridx = srow + l
                acc_a = acc_a + plsc.load_gather(alpha_v, [ridx])
                acc_aw = acc_aw + plsc.load_gather(awe_v, [ridx])
                return acc_a, acc_aw

            z = jnp.zeros((L16,), jnp.float32)
            acc_a, acc_aw = lax.fori_loop(0, S, s_body, (z, z))
            out_v[pl.ds(sg * L16, L16)] = acc_aw / jnp.maximum(acc_a, 1e-12)

        pltpu.sync_copy(out_v, out_hbm.at[pl.ds(wid * sent_w, sent_w)])

    return sck


def kernel(batch_word_idxs, word_embeddings, weights, attend_u):
    B, S = batch_word_idxs.shape
    V, D = word_embeddings.shape
    DP = ((D + L16 - 1) // L16) * L16
    idx_flat = batch_word_idxs.reshape(-1).astype(jnp.int32)
    table = word_embeddings.astype(jnp.float32)
    u_vec = attend_u.astype(jnp.float32)
    w_vec = weights.reshape(-1).astype(jnp.float32)
    u_pad = jnp.pad(u_vec, (0, DP - D))

    m1 = jnp.zeros((D, PK), jnp.float32)
    m1 = m1.at[:, 0].set(u_vec).at[:, 1].set(w_vec)
    m2 = jnp.zeros((D, PK), jnp.float32).at[:, 2].set(1.0)

    pk = _make_tc_scan(V, D)(table, m1, m2).reshape(-1)
    sck = _make_sc_kernel(B, S, D, DP, pk.shape[0])
    out = sck(idx_flat, pk, u_pad)
    return out.reshape(B, 1)


# speedup vs baseline: 2.8967x; 1.0184x over previous
import functools

import jax
import jax.numpy as jnp
from jax import lax
from jax.experimental import pallas as pl
from jax.experimental.pallas import tpu as pltpu
from jax.experimental.pallas import tpu_sc as plsc

NC = 2
NS = 16
NW = NC * NS
L16 = 16
PK = 128
BLK = 16384
BLK_LOG = 14


def _fsqrt(x):
    x = jnp.maximum(x, 1e-20)
    i = lax.bitcast_convert_type(x, jnp.int32)
    i = jnp.int32(0x5F3759DF) - lax.shift_right_logical(i, 1)
    y = lax.bitcast_convert_type(i, jnp.float32)
    for _ in range(3):
        y = y * (1.5 - 0.5 * x * y * y)
    return x * y


def _make_tc_scan(V, D):
    def body(tab_ref, m1_ref, m2_ref, out_ref):
        e = tab_ref[...]
        f = e * e
        s = (jnp.dot(e, m1_ref[...], preferred_element_type=jnp.float32)
             + jnp.dot(f, m2_ref[...], preferred_element_type=jnp.float32))
        out_ref[...] = s[:, 0:8].T

    grid = (V + BLK - 1) // BLK
    return pl.pallas_call(
        body,
        grid=(grid,),
        in_specs=[
            pl.BlockSpec((BLK, D), lambda i: (i, 0)),
            pl.BlockSpec((D, PK), lambda i: (0, 0)),
            pl.BlockSpec((D, PK), lambda i: (0, 0)),
        ],
        out_specs=pl.BlockSpec((8, BLK), lambda i: (i, 0)),
        out_shape=jax.ShapeDtypeStruct((grid * 8, BLK), jnp.float32),
    )


def _make_sc_kernel(B, S, D, DP, VP):
    rows_w = (B // NW) * S
    n_chunks = rows_w // 128
    sent_w = B // NW
    n_sgrp = sent_w // L16

    mesh = plsc.VectorSubcoreMesh(core_axis_name="c", subcore_axis_name="s")

    @functools.partial(
        pl.kernel,
        mesh=mesh,
        out_type=jax.ShapeDtypeStruct((B,), jnp.float32),
        compiler_params=pltpu.CompilerParams(
            use_tc_tiling_on_sc=False, needs_layout_passes=False),
        scratch_types=[
            pltpu.VMEM((n_chunks, 128), jnp.int32),
            pltpu.VMEM((3 * n_chunks, 128), jnp.int32),
            pltpu.VMEM((3 * rows_w,), jnp.float32),
            pltpu.VMEM((DP,), jnp.float32),
            pltpu.VMEM((rows_w,), jnp.float32),
            pltpu.VMEM((rows_w,), jnp.float32),
            pltpu.VMEM((sent_w,), jnp.float32),
            pltpu.SemaphoreType.DMA,
        ],
    )
    def sck(idx_hbm, pk_hbm, u_hbm, out_hbm,
            idx_v, gidx_v, pk_v, u_v, alpha_v, awe_v, out_v, sem):
        wid = lax.axis_index("s") * NC + lax.axis_index("c")

        for j in range(n_chunks):
            pltpu.sync_copy(
                idx_hbm.at[pl.ds(wid * rows_w + j * 128, 128)], idx_v.at[j])
        pltpu.sync_copy(u_hbm, u_v)

        lane = lax.iota(jnp.int32, L16)

        for j in range(n_chunks):
            for c in range(128 // L16):
                v = idx_v[j, pl.ds(c * L16, L16)]
                base = (lax.shift_left(
                    lax.shift_right_logical(v, BLK_LOG), BLK_LOG + 3)
                    + (v & (BLK - 1)))
                p = 3 * (j * 128 + c * L16 + lane)
                for t in range(3):
                    pt = p + t
                    plsc.store_scatter(
                        gidx_v,
                        [lax.shift_right_logical(pt, 7), pt & 127],
                        base + BLK * t)

        handles = [
            pltpu.async_copy(pk_hbm.at[gidx_v.at[j]],
                             pk_v.at[pl.ds(j * 128, 128)], sem)
            for j in range(3 * n_chunks)
        ]

        uacc = jnp.zeros((L16,), jnp.float32)
        for c in range(DP // L16):
            uc = u_v[pl.ds(c * L16, L16)]
            uacc = uacc + uc * uc
        u2 = jnp.sum(uacc)

        for h in handles:
            h.wait()

        lane3 = 3 * lane
        for g in range(rows_w // L16):
            base3 = lane3 + 3 * g * L16
            su = plsc.load_gather(pk_v, [base3])
            sw = plsc.load_gather(pk_v, [base3 + 1])
            se2 = plsc.load_gather(pk_v, [base3 + 2])
            d2 = (u2 - 2.0 * su) + se2
            alpha = jnp.exp(_fsqrt(d2))
            sl = pl.ds(g * L16, L16)
            alpha_v[sl] = alpha
            awe_v[sl] = alpha * sw

        for sg in range(n_sgrp):
            srow = (sg * L16 + lane) * S

            def s_body(l, carry, srow=srow):
                acc_a, acc_aw = carry
                ridx = srow + l
                acc_a = acc_a + plsc.load_gather(alpha_v, [ridx])
                acc_aw = acc_aw + plsc.load_gather(awe_v, [ridx])
                return acc_a, acc_aw

            z = jnp.zeros((L16,), jnp.float32)
            acc_a, acc_aw = lax.fori_loop(0, S, s_body, (z, z))
            out_v[pl.ds(sg * L16, L16)] = acc_aw / jnp.maximum(acc_a, 1e-12)

        pltpu.sync_copy(out_v, out_hbm.at[pl.ds(wid * sent_w, sent_w)])

    return sck


def kernel(batch_word_idxs, word_embeddings, weights, attend_u):
    B, S = batch_word_idxs.shape
    V, D = word_embeddings.shape
    DP = ((D + L16 - 1) // L16) * L16
    idx_flat = batch_word_idxs.reshape(-1).astype(jnp.int32)
    table = word_embeddings.astype(jnp.float32)
    u_vec = attend_u.astype(jnp.float32)
    w_vec = weights.reshape(-1).astype(jnp.float32)
    u_pad = jnp.pad(u_vec, (0, DP - D))

    m1 = jnp.zeros((D, PK), jnp.float32)
    m1 = m1.at[:, 0].set(u_vec).at[:, 1].set(w_vec)
    m2 = jnp.zeros((D, PK), jnp.float32).at[:, 2].set(1.0)

    pk = _make_tc_scan(V, D)(table, m1, m2).reshape(-1)
    sck = _make_sc_kernel(B, S, D, DP, pk.shape[0])
    out = sck(idx_flat, pk, u_pad)
    return out.reshape(B, 1)
